# Initial kernel scaffold; baseline (speedup 1.0000x reference)
import jax
import jax.numpy as jnp
from jax.experimental import pallas as pl


def kernel(x, edge_index, W1, W2):
    raise NotImplementedError("write your pallas kernel here")


# speedup vs baseline: 5.0710x; 5.0710x over previous
import functools

import jax
import jax.numpy as jnp
from jax import lax
from jax.experimental import pallas as pl
from jax.experimental.pallas import tpu as pltpu
from jax.experimental.pallas import tpu_sc as plsc

N_NODES = 10000
N_EDGES = 320000
D = 128

NC = 2
NS = 16
CHUNK = 80
EDGES_PER_SC = N_EDGES // NC
EDGES_PER_TILE = EDGES_PER_SC // NS
N_CHUNKS = EDGES_PER_TILE // CHUNK
N_PAD = 10240
ROWS_PER_TILE = N_PAD // NS


@functools.partial(
    pl.kernel,
    out_type=jax.ShapeDtypeStruct((NC, N_PAD, D), jnp.float32),
    mesh=plsc.VectorSubcoreMesh(core_axis_name="c", subcore_axis_name="s"),
    scratch_types=[
        pltpu.VMEM((CHUNK,), jnp.int32),
        pltpu.VMEM((CHUNK,), jnp.int32),
        pltpu.VMEM((CHUNK, D), jnp.float32),
        pltpu.VMEM_SHARED((N_PAD, D), jnp.float32),
        pltpu.SemaphoreType.DMA,
    ],
)
def _spmm(h_hbm, src_hbm, dst_hbm, zeros_hbm, out_hbm,
          src_v, dst_v, rows_v, acc_sh, sem):
    c = lax.axis_index("c")
    s = lax.axis_index("s")
    r0 = s * ROWS_PER_TILE
    pltpu.sync_copy(zeros_hbm.at[pl.ds(r0, ROWS_PER_TILE)],
                    acc_sh.at[pl.ds(r0, ROWS_PER_TILE)])
    plsc.subcore_barrier()

    e0 = c * EDGES_PER_SC + s * EDGES_PER_TILE

    def body(i, carry):
        base = e0 + i * CHUNK
        pltpu.sync_copy(src_hbm.at[pl.ds(base, CHUNK)], src_v)
        pltpu.sync_copy(dst_hbm.at[pl.ds(base, CHUNK)], dst_v)
        pltpu.async_copy(h_hbm.at[src_v], rows_v, sem).wait()
        pltpu.sync_copy(rows_v, acc_sh.at[dst_v], add=True)
        return carry

    lax.fori_loop(0, N_CHUNKS, body, 0)
    plsc.subcore_barrier()
    pltpu.sync_copy(acc_sh.at[pl.ds(r0, ROWS_PER_TILE)],
                    out_hbm.at[c, pl.ds(r0, ROWS_PER_TILE)])


def _mm_body(x_ref, w_ref, o_ref):
    o_ref[...] = jnp.dot(x_ref[...], w_ref[...],
                         preferred_element_type=jnp.float32)


_mm = pl.pallas_call(
    _mm_body,
    out_shape=jax.ShapeDtypeStruct((N_NODES, D), jnp.float32),
)


def _addrelu_body(p_ref, o_ref):
    o_ref[...] = jnp.maximum(p_ref[0] + p_ref[1], 0.0)


_addrelu = pl.pallas_call(
    _addrelu_body,
    out_shape=jax.ShapeDtypeStruct((N_NODES, D), jnp.float32),
)


def _final_body(p_ref, w_ref, o_ref):
    g = jnp.dot(p_ref[0] + p_ref[1], w_ref[...],
                preferred_element_type=jnp.float32)
    m = jnp.max(g, axis=1, keepdims=True)
    o_ref[...] = (g - m) - jnp.log(jnp.sum(jnp.exp(g - m), axis=1,
                                           keepdims=True))


_final = pl.pallas_call(
    _final_body,
    out_shape=jax.ShapeDtypeStruct((N_NODES, D), jnp.float32),
)


def kernel(x, edge_index, W1, W2):
    src = edge_index[1].astype(jnp.int32)
    dst = edge_index[0].astype(jnp.int32)
    zeros = jnp.zeros((N_PAD, D), jnp.float32)
    h1 = _mm(x, W1)
    p1 = _spmm(h1, src, dst, zeros)[:, :N_NODES]
    r = _addrelu(p1)
    p2 = _spmm(r, src, dst, zeros)[:, :N_NODES]
    return _final(p2, W2)


# speedup vs baseline: 9.8236x; 1.9372x over previous
import functools

import jax
import jax.numpy as jnp
from jax import lax
from jax.experimental import pallas as pl
from jax.experimental.pallas import tpu as pltpu
from jax.experimental.pallas import tpu_sc as plsc

N_NODES = 10000
N_EDGES = 320000
D = 128

NC = 2
NS = 16
CHUNK = 80
EDGES_PER_SC = N_EDGES // NC
EDGES_PER_TILE = EDGES_PER_SC // NS
N_CHUNKS = EDGES_PER_TILE // CHUNK
N_PAD = 10240
ROWS_PER_TILE = N_PAD // NS


NBUF = 4
N_MAIN = (N_CHUNKS - 1) // NBUF * NBUF
N_GROUPS = N_MAIN // NBUF


@functools.partial(
    pl.kernel,
    out_type=jax.ShapeDtypeStruct((NC, N_PAD, D), jnp.float32),
    mesh=plsc.VectorSubcoreMesh(core_axis_name="c", subcore_axis_name="s"),
    scratch_types=[
        pltpu.VMEM((NBUF, 2, CHUNK), jnp.int32),
        pltpu.VMEM((NBUF, CHUNK, D), jnp.float32),
        pltpu.VMEM_SHARED((N_PAD, D), jnp.float32),
        pltpu.SemaphoreType.DMA((NBUF,)),
        pltpu.SemaphoreType.DMA((NBUF,)),
        pltpu.SemaphoreType.DMA((NBUF,)),
    ],
)
def _spmm(h_hbm, idx4_hbm, zeros_hbm, out_hbm,
          idx_v, rows_v, acc_sh, isem, gsem, ssem):
    c = lax.axis_index("c")
    s = lax.axis_index("s")
    w = c * NS + s
    r0 = s * ROWS_PER_TILE

    def idx_start(j, b):
        pltpu.async_copy(idx4_hbm.at[w, j], idx_v.at[b], isem.at[b])

    def idx_wait(j, b):
        pltpu.make_async_copy(idx4_hbm.at[w, j], idx_v.at[b],
                              isem.at[b]).wait()

    def gather_start(b):
        pltpu.async_copy(h_hbm.at[idx_v.at[b, 0]], rows_v.at[b], gsem.at[b])

    def gather_wait(b):
        pltpu.make_async_copy(h_hbm.at[idx_v.at[b, 0]], rows_v.at[b],
                              gsem.at[b]).wait()

    def scatter_start(b):
        pltpu.async_copy(rows_v.at[b], acc_sh.at[idx_v.at[b, 1]], ssem.at[b],
                         add=True)

    def scatter_wait(b):
        pltpu.make_async_copy(rows_v.at[b], acc_sh.at[idx_v.at[b, 1]],
                              ssem.at[b]).wait()

    for b in range(NBUF):
        idx_start(b, b)
    pltpu.sync_copy(zeros_hbm.at[pl.ds(r0, ROWS_PER_TILE)],
                    acc_sh.at[pl.ds(r0, ROWS_PER_TILE)])
    plsc.subcore_barrier()

    def group(g, carry):
        for b in range(NBUF):
            idx_wait(g * NBUF + b, b)
            gather_start(b)
        for b in range(NBUF):
            gather_wait(b)
            scatter_start(b)
        for b in range(NBUF):
            scatter_wait(b)
            idx_start(g * NBUF + b + NBUF, b)
        return carry

    lax.fori_loop(0, N_GROUPS - 1, group, 0)

    gl = N_GROUPS - 1
    for b in range(NBUF):
        idx_wait(gl * NBUF + b, b)
        gather_start(b)
    for b in range(NBUF):
        gather_wait(b)
        scatter_start(b)
    for b in range(NBUF):
        scatter_wait(b)

    jt = N_CHUNKS - 1
    idx_start(jt, 0)
    idx_wait(jt, 0)
    gather_start(0)
    gather_wait(0)
    scatter_start(0)
    scatter_wait(0)

    plsc.subcore_barrier()
    pltpu.sync_copy(acc_sh.at[pl.ds(r0, ROWS_PER_TILE)],
                    out_hbm.at[c, pl.ds(r0, ROWS_PER_TILE)])


def _mm_body(x_ref, w_ref, o_ref):
    o_ref[...] = jnp.dot(x_ref[...], w_ref[...],
                         preferred_element_type=jnp.float32)


_mm = pl.pallas_call(
    _mm_body,
    out_shape=jax.ShapeDtypeStruct((N_NODES, D), jnp.float32),
)


def _addrelu_body(p_ref, o_ref):
    o_ref[...] = jnp.maximum(p_ref[0] + p_ref[1], 0.0)


_addrelu = pl.pallas_call(
    _addrelu_body,
    out_shape=jax.ShapeDtypeStruct((N_NODES, D), jnp.float32),
)


def _final_body(p_ref, w_ref, o_ref):
    g = jnp.dot(p_ref[0] + p_ref[1], w_ref[...],
                preferred_element_type=jnp.float32)
    m = jnp.max(g, axis=1, keepdims=True)
    o_ref[...] = (g - m) - jnp.log(jnp.sum(jnp.exp(g - m), axis=1,
                                           keepdims=True))


_final = pl.pallas_call(
    _final_body,
    out_shape=jax.ShapeDtypeStruct((N_NODES, D), jnp.float32),
)


def kernel(x, edge_index, W1, W2):
    src3 = edge_index[1].astype(jnp.int32).reshape(NC * NS, N_CHUNKS, CHUNK)
    dst3 = edge_index[0].astype(jnp.int32).reshape(NC * NS, N_CHUNKS, CHUNK)
    idx4 = jnp.stack([src3, dst3], axis=2)
    zeros = jnp.zeros((N_PAD, D), jnp.float32)
    h1 = _mm(x, W1)
    p1 = _spmm(h1, idx4, zeros)[:, :N_NODES]
    r = _addrelu(p1)
    p2 = _spmm(r, idx4, zeros)[:, :N_NODES]
    return _final(p2, W2)


# speedup vs baseline: 10.4851x; 1.0673x over previous
import functools

import jax
import jax.numpy as jnp
from jax import lax
from jax.experimental import pallas as pl
from jax.experimental.pallas import tpu as pltpu
from jax.experimental.pallas import tpu_sc as plsc

N_NODES = 10000
N_EDGES = 320000
D = 128

NC = 2
NS = 16
CHUNK = 80
EDGES_PER_SC = N_EDGES // NC
EDGES_PER_TILE = EDGES_PER_SC // NS
N_CHUNKS = EDGES_PER_TILE // CHUNK
N_PAD = 10240
ROWS_PER_TILE = N_PAD // NS


NBUF = 4
N_MAIN = (N_CHUNKS - 1) // NBUF * NBUF
N_GROUPS = N_MAIN // NBUF


@functools.partial(
    pl.kernel,
    out_type=jax.ShapeDtypeStruct((NC, N_PAD, D), jnp.float32),
    mesh=plsc.VectorSubcoreMesh(core_axis_name="c", subcore_axis_name="s"),
    scratch_types=[
        pltpu.VMEM((NBUF, 2, CHUNK), jnp.int32),
        pltpu.VMEM((NBUF, CHUNK, D), jnp.float32),
        pltpu.VMEM_SHARED((N_PAD, D), jnp.float32),
        pltpu.SemaphoreType.DMA((NBUF,)),
        pltpu.SemaphoreType.DMA((NBUF,)),
        pltpu.SemaphoreType.DMA((NBUF,)),
    ],
)
def _spmm(h_hbm, idx4_hbm, out_hbm,
          idx_v, rows_v, acc_sh, isem, gsem, ssem):
    c = lax.axis_index("c")
    s = lax.axis_index("s")
    w = c * NS + s
    r0 = s * ROWS_PER_TILE

    def idx_start(j, b):
        pltpu.async_copy(idx4_hbm.at[w, j], idx_v.at[b], isem.at[b])

    def idx_wait(j, b):
        pltpu.make_async_copy(idx4_hbm.at[w, j], idx_v.at[b],
                              isem.at[b]).wait()

    def gather_start(b):
        pltpu.async_copy(h_hbm.at[idx_v.at[b, 0]], rows_v.at[b], gsem.at[b])

    def gather_wait(b):
        pltpu.make_async_copy(h_hbm.at[idx_v.at[b, 0]], rows_v.at[b],
                              gsem.at[b]).wait()

    def scatter_start(b):
        pltpu.async_copy(rows_v.at[b], acc_sh.at[idx_v.at[b, 1]], ssem.at[b],
                         add=True)

    def scatter_wait(b):
        pltpu.make_async_copy(rows_v.at[b], acc_sh.at[idx_v.at[b, 1]],
                              ssem.at[b]).wait()

    for b in range(NBUF):
        idx_start(b, b)
    for i in range(CHUNK):
        for k in range(D // 16):
            rows_v[0, i, pl.ds(k * 16, 16)] = jnp.zeros((16,), jnp.float32)
    zcopies = [
        pltpu.make_async_copy(
            rows_v.at[0], acc_sh.at[pl.ds(r0 + t * CHUNK, CHUNK)], ssem.at[0])
        for t in range(ROWS_PER_TILE // CHUNK)
    ]
    for zc in zcopies:
        zc.start()
    for zc in zcopies:
        zc.wait()
    plsc.subcore_barrier()

    def group(g, carry):
        for b in range(NBUF):
            idx_wait(g * NBUF + b, b)
            gather_start(b)
        for b in range(NBUF):
            gather_wait(b)
            scatter_start(b)
        for b in range(NBUF):
            scatter_wait(b)
            idx_start(g * NBUF + b + NBUF, b)
        return carry

    lax.fori_loop(0, N_GROUPS - 1, group, 0)

    gl = N_GROUPS - 1
    for b in range(NBUF):
        idx_wait(gl * NBUF + b, b)
        gather_start(b)
    for b in range(NBUF):
        gather_wait(b)
        scatter_start(b)
    for b in range(NBUF):
        scatter_wait(b)

    jt = N_CHUNKS - 1
    idx_start(jt, 0)
    idx_wait(jt, 0)
    gather_start(0)
    gather_wait(0)
    scatter_start(0)
    scatter_wait(0)

    plsc.subcore_barrier()
    pltpu.sync_copy(acc_sh.at[pl.ds(r0, ROWS_PER_TILE)],
                    out_hbm.at[c, pl.ds(r0, ROWS_PER_TILE)])


def _mm_body(x_ref, w_ref, o_ref):
    o_ref[...] = jnp.dot(x_ref[...], w_ref[...],
                         preferred_element_type=jnp.float32)


_mm = pl.pallas_call(
    _mm_body,
    out_shape=jax.ShapeDtypeStruct((N_NODES, D), jnp.float32),
)


def _addrelu_body(p_ref, o_ref):
    o_ref[...] = jnp.maximum(p_ref[0] + p_ref[1], 0.0)


_addrelu = pl.pallas_call(
    _addrelu_body,
    out_shape=jax.ShapeDtypeStruct((N_PAD, D), jnp.float32),
)


def _final_body(p_ref, w_ref, o_ref):
    p = p_ref[0, pl.ds(0, N_NODES)] + p_ref[1, pl.ds(0, N_NODES)]
    g = jnp.dot(p, w_ref[...], preferred_element_type=jnp.float32)
    m = jnp.max(g, axis=1, keepdims=True)
    o_ref[...] = (g - m) - jnp.log(jnp.sum(jnp.exp(g - m), axis=1,
                                           keepdims=True))


_final = pl.pallas_call(
    _final_body,
    out_shape=jax.ShapeDtypeStruct((N_NODES, D), jnp.float32),
)


def kernel(x, edge_index, W1, W2):
    src3 = edge_index[1].astype(jnp.int32).reshape(NC * NS, N_CHUNKS, CHUNK)
    dst3 = edge_index[0].astype(jnp.int32).reshape(NC * NS, N_CHUNKS, CHUNK)
    idx4 = jnp.stack([src3, dst3], axis=2)
    h1 = _mm(x, W1)
    p1 = _spmm(h1, idx4)
    r = _addrelu(p1)
    p2 = _spmm(r, idx4)
    return _final(p2, W2)
